# baseline (speedup 1.0000x reference)
import functools

import jax
import jax.numpy as jnp
from jax import lax
from jax.experimental import pallas as pl
from jax.experimental.pallas import tpu as pltpu
from jax.experimental.pallas import tpu_sc as plsc

N_NODES = 10000
N_PAD = 10240
N_EDGES = 320000
NC = 2
NS = 16
NW = NC * NS
CHUNK = 128
N_CHUNKS = 80
E_PER_W = N_CHUNKS * CHUNK
ROWS_PER_TILE = N_PAD // NS
ZROWS = 128

_MESH = dict(core_axis_name="c", subcore_axis_name="s")
_SC_PARAMS = pltpu.CompilerParams(use_tc_tiling_on_sc=False)


def _wid():
    return lax.axis_index("s") * NC + lax.axis_index("c")


def _zero_fill(zbuf, ncols):
    zero = jnp.zeros((16,), jnp.float32)

    def body(j, _):
        for c in range(ncols // 16):
            zbuf[j, pl.ds(c * 16, 16)] = zero
        return 0

    lax.fori_loop(0, ZROWS, body, 0)


def _zero_acc_stripe(zbuf, acc_sh, sid):
    base = sid * ROWS_PER_TILE
    for k in range(ROWS_PER_TILE // ZROWS):
        pltpu.sync_copy(zbuf, acc_sh.at[pl.ds(base + k * ZROWS, ZROWS)])


def _write_stripe(acc_sh, out_hbm, cid, sid, d):
    base = sid * ROWS_PER_TILE
    pltpu.sync_copy(acc_sh.at[pl.ds(base, ROWS_PER_TILE)],
                    out_hbm.at[pl.ds(base, ROWS_PER_TILE), pl.ds(d * cid, d)])


@functools.partial(
    pl.kernel,
    mesh=plsc.VectorSubcoreMesh(**_MESH),
    out_type=jax.ShapeDtypeStruct((N_PAD, 128), jnp.float32),
    scratch_types=[
        pltpu.VMEM((N_CHUNKS, CHUNK), jnp.int32),
        pltpu.VMEM((CHUNK, 16), jnp.float32),
        pltpu.VMEM((ZROWS, 16), jnp.float32),
        pltpu.VMEM_SHARED((N_PAD, 16), jnp.float32),
    ],
    compiler_params=_SC_PARAMS,
)
def _deg(edges_hbm, out_hbm, dst_v, ones_v, zbuf, acc_sh):
    cid = lax.axis_index("c")
    sid = lax.axis_index("s")
    pltpu.sync_copy(edges_hbm.at[1, _wid()], dst_v)

    one = jnp.full((16,), 1.0, jnp.float32)

    def fill(j, _):
        ones_v[j, :] = one
        return 0

    lax.fori_loop(0, CHUNK, fill, 0)
    _zero_fill(zbuf, 16)
    _zero_acc_stripe(zbuf, acc_sh, sid)
    plsc.subcore_barrier()

    def chunk(j, _):
        pltpu.sync_copy(ones_v, acc_sh.at[dst_v.at[j]], add=True)
        return 0

    lax.fori_loop(0, N_CHUNKS, chunk, 0)
    plsc.subcore_barrier()
    _write_stripe(acc_sh, out_hbm, cid, sid, 16)


def _make_agg(d):
    @functools.partial(
        pl.kernel,
        mesh=plsc.VectorSubcoreMesh(**_MESH),
        out_type=jax.ShapeDtypeStruct((N_PAD, 128), jnp.float32),
        scratch_types=[
            pltpu.VMEM((N_CHUNKS, CHUNK), jnp.int32),
            pltpu.VMEM((N_CHUNKS, CHUNK), jnp.int32),
            pltpu.VMEM((CHUNK, d), jnp.float32),
            pltpu.VMEM((CHUNK, d), jnp.float32),
            pltpu.VMEM((ZROWS, d), jnp.float32),
            pltpu.VMEM_SHARED((N_PAD, d), jnp.float32),
            pltpu.SemaphoreType.DMA,
            pltpu.SemaphoreType.DMA,
        ],
        compiler_params=_SC_PARAMS,
    )
    def agg(hs_hbm, edges_hbm, out_hbm, src_v, dst_v, rows0, rows1,
            zbuf, acc_sh, sem0, sem1):
        cid = lax.axis_index("c")
        sid = lax.axis_index("s")
        wid = _wid()
        pltpu.sync_copy(edges_hbm.at[0, wid], src_v)
        pltpu.sync_copy(edges_hbm.at[1, wid], dst_v)
        _zero_fill(zbuf, d)
        _zero_acc_stripe(zbuf, acc_sh, sid)
        plsc.subcore_barrier()

        def gather(j, buf, sem):
            pltpu.async_copy(hs_hbm.at[src_v.at[j]], buf, sem)

        def wait(j, buf, sem):
            pltpu.make_async_copy(hs_hbm.at[src_v.at[j]], buf, sem).wait()

        def scat(j, buf):
            pltpu.sync_copy(buf, acc_sh.at[dst_v.at[j]], add=True)

        gather(0, rows0, sem0)
        gather(1, rows1, sem1)

        def pair(jj, _):
            j0 = 2 * jj
            j1 = j0 + 1
            wait(j0, rows0, sem0)
            scat(j0, rows0)
            gather(j0 + 2, rows0, sem0)
            wait(j1, rows1, sem1)
            scat(j1, rows1)
            gather(j1 + 2, rows1, sem1)
            return 0

        lax.fori_loop(0, N_CHUNKS // 2 - 1, pair, 0)
        t0 = N_CHUNKS - 2
        wait(t0, rows0, sem0)
        scat(t0, rows0)
        wait(t0 + 1, rows1, sem1)
        scat(t0 + 1, rows1)

        plsc.subcore_barrier()
        _write_stripe(acc_sh, out_hbm, cid, sid, d)

    return agg


_agg64 = _make_agg(64)
_agg32 = _make_agg(32)


BLK = 2000
_GRID = N_NODES // BLK


def _row_spec(cols):
    return pl.BlockSpec((BLK, cols), lambda i: (i, 0))


def _full_spec(r, c):
    return pl.BlockSpec((r, c), lambda i: (0, 0))


def _mm1_body(x_ref, w1_ref, degp_ref, hs_ref, dinv_ref):
    deg = degp_ref[:, 0:1] + degp_ref[:, 16:17] + 1.0
    dinv = lax.rsqrt(deg)
    h = jnp.dot(x_ref[...], w1_ref[...], preferred_element_type=jnp.float32)
    hs_ref[...] = h * dinv
    dinv_ref[...] = dinv


def _mm2_body(aggp_ref, hs1_ref, dinv_ref, b1_ref, w2_ref, hs2_ref):
    dinv = dinv_ref[...]
    agg = aggp_ref[:, 0:64] + aggp_ref[:, 64:128] + hs1_ref[...]
    x2 = jnp.maximum(dinv * agg + b1_ref[...], 0.0)
    hs2_ref[...] = dinv * jnp.dot(x2, w2_ref[...],
                                  preferred_element_type=jnp.float32)


def _mm3_body(aggp_ref, hs2_ref, dinv_ref, b2_ref, fcw_ref, fcb_ref,
              emb_ref, log_ref):
    dinv = dinv_ref[...]
    agg = aggp_ref[:, 0:32] + aggp_ref[:, 32:64] + hs2_ref[...]
    emb = jnp.maximum(dinv * agg + b2_ref[...], 0.0)
    emb_ref[...] = emb
    log_ref[...] = jnp.dot(emb, fcw_ref[...],
                           preferred_element_type=jnp.float32) + fcb_ref[...]


@jax.jit
def kernel(x, edge_index, W1, b1, W2, b2, fcW, fcb):
    n_real_w = N_EDGES // NW
    n_pad_w = E_PER_W - n_real_w
    ei = edge_index.astype(jnp.int32).reshape(2, NW, n_real_w)
    pad_src = jnp.zeros((NW, n_pad_w), jnp.int32)
    pad_dst = jnp.broadcast_to(
        N_NODES + jnp.arange(n_pad_w, dtype=jnp.int32), (NW, n_pad_w))
    edges = jnp.stack([
        jnp.concatenate([ei[0], pad_src], axis=1),
        jnp.concatenate([ei[1], pad_dst], axis=1),
    ]).reshape(2, NW, N_CHUNKS, CHUNK)

    degp = _deg(edges)

    hs1, dinv = pl.pallas_call(
        _mm1_body,
        grid=(_GRID,),
        in_specs=[_row_spec(128), _full_spec(128, 64), _row_spec(128)],
        out_specs=[_row_spec(64), _row_spec(1)],
        out_shape=[
            jax.ShapeDtypeStruct((N_NODES, 64), jnp.float32),
            jax.ShapeDtypeStruct((N_NODES, 1), jnp.float32),
        ],
    )(x, W1, degp)

    aggp1 = _agg64(hs1, edges)

    hs2 = pl.pallas_call(
        _mm2_body,
        grid=(_GRID,),
        in_specs=[_row_spec(128), _row_spec(64), _row_spec(1),
                  _full_spec(1, 64), _full_spec(64, 32)],
        out_specs=_row_spec(32),
        out_shape=jax.ShapeDtypeStruct((N_NODES, 32), jnp.float32),
    )(aggp1, hs1, dinv, b1.reshape(1, -1), W2)

    aggp2 = _agg32(hs2, edges)

    emb, logits = pl.pallas_call(
        _mm3_body,
        grid=(_GRID,),
        in_specs=[_row_spec(128), _row_spec(32), _row_spec(1),
                  _full_spec(1, 32), _full_spec(32, 40), _full_spec(1, 40)],
        out_specs=[_row_spec(32), _row_spec(40)],
        out_shape=[
            jax.ShapeDtypeStruct((N_NODES, 32), jnp.float32),
            jax.ShapeDtypeStruct((N_NODES, 40), jnp.float32),
        ],
    )(aggp2, hs2, dinv, b2.reshape(1, -1), fcW, fcb.reshape(1, -1))

    return emb, logits


# speedup vs baseline: 1.0006x; 1.0006x over previous
import functools

import jax
import jax.numpy as jnp
from jax import lax
from jax.experimental import pallas as pl
from jax.experimental.pallas import tpu as pltpu
from jax.experimental.pallas import tpu_sc as plsc

N_NODES = 10000
N_PAD = 10240
N_EDGES = 320000
NC = 2
NS = 16
NW = NC * NS
CHUNK = 128
N_CHUNKS = 80
E_PER_W = N_CHUNKS * CHUNK
N_PAD_W = E_PER_W - N_EDGES // NW
N_ACC = N_PAD + NW * N_PAD_W
ROWS_PER_TILE = N_PAD // NS
ZROWS = 128

_MESH = dict(core_axis_name="c", subcore_axis_name="s")
_SC_PARAMS = pltpu.CompilerParams(use_tc_tiling_on_sc=False)


def _wid():
    return lax.axis_index("s") * NC + lax.axis_index("c")


def _zero_fill(zbuf, ncols):
    zero = jnp.zeros((16,), jnp.float32)

    def body(j, _):
        for c in range(ncols // 16):
            zbuf[j, pl.ds(c * 16, 16)] = zero
        return 0

    lax.fori_loop(0, ZROWS, body, 0)


def _zero_acc_stripe(zbuf, acc_sh, sid):
    base = sid * ROWS_PER_TILE
    for k in range(ROWS_PER_TILE // ZROWS):
        pltpu.sync_copy(zbuf, acc_sh.at[pl.ds(base + k * ZROWS, ZROWS)])


def _write_stripe(acc_sh, out_hbm, cid, sid, d):
    base = sid * ROWS_PER_TILE
    pltpu.sync_copy(acc_sh.at[pl.ds(base, ROWS_PER_TILE)],
                    out_hbm.at[pl.ds(base, ROWS_PER_TILE), pl.ds(d * cid, d)])


@functools.partial(
    pl.kernel,
    mesh=plsc.VectorSubcoreMesh(**_MESH),
    out_type=jax.ShapeDtypeStruct((N_PAD, 128), jnp.float32),
    scratch_types=[
        pltpu.VMEM((N_CHUNKS, CHUNK), jnp.int32),
        pltpu.VMEM((CHUNK, 16), jnp.float32),
        pltpu.VMEM((ZROWS, 16), jnp.float32),
        pltpu.VMEM_SHARED((N_ACC, 16), jnp.float32),
    ],
    compiler_params=_SC_PARAMS,
)
def _deg(edges_hbm, out_hbm, dst_v, ones_v, zbuf, acc_sh):
    cid = lax.axis_index("c")
    sid = lax.axis_index("s")
    pltpu.sync_copy(edges_hbm.at[1, _wid()], dst_v)

    one = jnp.full((16,), 1.0, jnp.float32)

    def fill(j, _):
        ones_v[j, :] = one
        return 0

    lax.fori_loop(0, CHUNK, fill, 0)
    _zero_fill(zbuf, 16)
    _zero_acc_stripe(zbuf, acc_sh, sid)
    plsc.subcore_barrier()

    def chunk(j, _):
        pltpu.sync_copy(ones_v, acc_sh.at[dst_v.at[j]], add=True)
        return 0

    lax.fori_loop(0, N_CHUNKS, chunk, 0)
    plsc.subcore_barrier()
    _write_stripe(acc_sh, out_hbm, cid, sid, 16)


def _make_agg(d):
    @functools.partial(
        pl.kernel,
        mesh=plsc.VectorSubcoreMesh(**_MESH),
        out_type=jax.ShapeDtypeStruct((N_PAD, 128), jnp.float32),
        scratch_types=[
            pltpu.VMEM((N_CHUNKS, CHUNK), jnp.int32),
            pltpu.VMEM((N_CHUNKS, CHUNK), jnp.int32),
            pltpu.VMEM((CHUNK, d), jnp.float32),
            pltpu.VMEM((CHUNK, d), jnp.float32),
            pltpu.VMEM((ZROWS, d), jnp.float32),
            pltpu.VMEM_SHARED((N_ACC, d), jnp.float32),
            pltpu.SemaphoreType.DMA,
            pltpu.SemaphoreType.DMA,
        ],
        compiler_params=_SC_PARAMS,
    )
    def agg(hs_hbm, edges_hbm, out_hbm, src_v, dst_v, rows0, rows1,
            zbuf, acc_sh, sem0, sem1):
        cid = lax.axis_index("c")
        sid = lax.axis_index("s")
        wid = _wid()
        pltpu.sync_copy(edges_hbm.at[0, wid], src_v)
        pltpu.sync_copy(edges_hbm.at[1, wid], dst_v)
        _zero_fill(zbuf, d)
        _zero_acc_stripe(zbuf, acc_sh, sid)
        plsc.subcore_barrier()

        def gather(j, buf, sem):
            pltpu.async_copy(hs_hbm.at[src_v.at[j]], buf, sem)

        def wait(j, buf, sem):
            pltpu.make_async_copy(hs_hbm.at[src_v.at[j]], buf, sem).wait()

        def scat(j, buf):
            pltpu.sync_copy(buf, acc_sh.at[dst_v.at[j]], add=True)

        gather(0, rows0, sem0)
        gather(1, rows1, sem1)

        def pair(jj, _):
            j0 = 2 * jj
            j1 = j0 + 1
            wait(j0, rows0, sem0)
            scat(j0, rows0)
            gather(j0 + 2, rows0, sem0)
            wait(j1, rows1, sem1)
            scat(j1, rows1)
            gather(j1 + 2, rows1, sem1)
            return 0

        lax.fori_loop(0, N_CHUNKS // 2 - 1, pair, 0)
        t0 = N_CHUNKS - 2
        wait(t0, rows0, sem0)
        scat(t0, rows0)
        wait(t0 + 1, rows1, sem1)
        scat(t0 + 1, rows1)

        plsc.subcore_barrier()
        _write_stripe(acc_sh, out_hbm, cid, sid, d)

    return agg


_agg64 = _make_agg(64)
_agg32 = _make_agg(32)


BLK = 2000
_GRID = N_NODES // BLK


def _row_spec(cols):
    return pl.BlockSpec((BLK, cols), lambda i: (i, 0))


def _full_spec(r, c):
    return pl.BlockSpec((r, c), lambda i: (0, 0))


def _mm1_body(x_ref, w1_ref, degp_ref, hs_ref, dinv_ref):
    deg = degp_ref[:, 0:1] + degp_ref[:, 16:17] + 1.0
    dinv = lax.rsqrt(deg)
    h = jnp.dot(x_ref[...], w1_ref[...], preferred_element_type=jnp.float32)
    hs_ref[...] = h * dinv
    dinv_ref[...] = dinv


def _mm2_body(aggp_ref, hs1_ref, dinv_ref, b1_ref, w2_ref, hs2_ref):
    dinv = dinv_ref[...]
    agg = aggp_ref[:, 0:64] + aggp_ref[:, 64:128] + hs1_ref[...]
    x2 = jnp.maximum(dinv * agg + b1_ref[...], 0.0)
    hs2_ref[...] = dinv * jnp.dot(x2, w2_ref[...],
                                  preferred_element_type=jnp.float32)


def _mm3_body(aggp_ref, hs2_ref, dinv_ref, b2_ref, fcw_ref, fcb_ref,
              emb_ref, log_ref):
    dinv = dinv_ref[...]
    agg = aggp_ref[:, 0:32] + aggp_ref[:, 32:64] + hs2_ref[...]
    emb = jnp.maximum(dinv * agg + b2_ref[...], 0.0)
    emb_ref[...] = emb
    log_ref[...] = jnp.dot(emb, fcw_ref[...],
                           preferred_element_type=jnp.float32) + fcb_ref[...]


@jax.jit
def kernel(x, edge_index, W1, b1, W2, b2, fcW, fcb):
    ei = edge_index.astype(jnp.int32).reshape(2, NW, N_EDGES // NW)
    pad_src = jnp.zeros((NW, N_PAD_W), jnp.int32)
    pad_dst = (N_PAD
               + jnp.arange(NW, dtype=jnp.int32)[:, None] * N_PAD_W
               + jnp.arange(N_PAD_W, dtype=jnp.int32)[None, :])
    edges = jnp.stack([
        jnp.concatenate([ei[0], pad_src], axis=1),
        jnp.concatenate([ei[1], pad_dst], axis=1),
    ]).reshape(2, NW, N_CHUNKS, CHUNK)

    degp = _deg(edges)

    hs1, dinv = pl.pallas_call(
        _mm1_body,
        grid=(_GRID,),
        in_specs=[_row_spec(128), _full_spec(128, 64), _row_spec(128)],
        out_specs=[_row_spec(64), _row_spec(1)],
        out_shape=[
            jax.ShapeDtypeStruct((N_NODES, 64), jnp.float32),
            jax.ShapeDtypeStruct((N_NODES, 1), jnp.float32),
        ],
    )(x, W1, degp)

    aggp1 = _agg64(hs1, edges)

    hs2 = pl.pallas_call(
        _mm2_body,
        grid=(_GRID,),
        in_specs=[_row_spec(128), _row_spec(64), _row_spec(1),
                  _full_spec(1, 64), _full_spec(64, 32)],
        out_specs=_row_spec(32),
        out_shape=jax.ShapeDtypeStruct((N_NODES, 32), jnp.float32),
    )(aggp1, hs1, dinv, b1.reshape(1, -1), W2)

    aggp2 = _agg32(hs2, edges)

    emb, logits = pl.pallas_call(
        _mm3_body,
        grid=(_GRID,),
        in_specs=[_row_spec(128), _row_spec(32), _row_spec(1),
                  _full_spec(1, 32), _full_spec(32, 40), _full_spec(1, 40)],
        out_specs=[_row_spec(32), _row_spec(40)],
        out_shape=[
            jax.ShapeDtypeStruct((N_NODES, 32), jnp.float32),
            jax.ShapeDtypeStruct((N_NODES, 40), jnp.float32),
        ],
    )(aggp2, hs2, dinv, b2.reshape(1, -1), fcW, fcb.reshape(1, -1))

    return emb, logits


# speedup vs baseline: 1.9298x; 1.9286x over previous
import functools

import jax
import jax.numpy as jnp
from jax import lax
from jax.experimental import pallas as pl
from jax.experimental.pallas import tpu as pltpu
from jax.experimental.pallas import tpu_sc as plsc

N_NODES = 10000
N_PAD = 10240
N_EDGES = 320000
NC = 2
NS = 16
NW = NC * NS
CHUNK = 128
N_CHUNKS = 80
E_PER_W = N_CHUNKS * CHUNK
N_PAD_W = E_PER_W - N_EDGES // NW
N_ACC = N_PAD + NW * N_PAD_W
ROWS_PER_TILE = N_PAD // NS
ZROWS = 128

_MESH = dict(core_axis_name="c", subcore_axis_name="s")
_SC_PARAMS = pltpu.CompilerParams(use_tc_tiling_on_sc=False)


def _wid():
    return lax.axis_index("s") * NC + lax.axis_index("c")


def _zero_fill(zbuf, ncols):
    zero = jnp.zeros((16,), jnp.float32)

    def body(j, _):
        for c in range(ncols // 16):
            zbuf[j, pl.ds(c * 16, 16)] = zero
        return 0

    lax.fori_loop(0, ZROWS, body, 0)


def _zero_acc_stripe(zbuf, acc_sh, sid):
    base = sid * ROWS_PER_TILE
    for k in range(ROWS_PER_TILE // ZROWS):
        pltpu.sync_copy(zbuf, acc_sh.at[pl.ds(base + k * ZROWS, ZROWS)])


def _write_stripe(acc_sh, out_hbm, cid, sid, d):
    base = sid * ROWS_PER_TILE
    pltpu.sync_copy(acc_sh.at[pl.ds(base, ROWS_PER_TILE)],
                    out_hbm.at[pl.ds(base, ROWS_PER_TILE), pl.ds(d * cid, d)])


@functools.partial(
    pl.kernel,
    mesh=plsc.VectorSubcoreMesh(**_MESH),
    out_type=jax.ShapeDtypeStruct((N_PAD, 128), jnp.float32),
    scratch_types=[
        pltpu.VMEM((N_CHUNKS, CHUNK), jnp.int32),
        pltpu.VMEM((CHUNK, 16), jnp.float32),
        pltpu.VMEM((ZROWS, 16), jnp.float32),
        pltpu.VMEM_SHARED((N_ACC, 16), jnp.float32),
    ],
    compiler_params=_SC_PARAMS,
)
def _deg(edges_hbm, out_hbm, dst_v, ones_v, zbuf, acc_sh):
    cid = lax.axis_index("c")
    sid = lax.axis_index("s")
    pltpu.sync_copy(edges_hbm.at[1, _wid()], dst_v)

    one = jnp.full((16,), 1.0, jnp.float32)

    def fill(j, _):
        ones_v[j, :] = one
        return 0

    lax.fori_loop(0, CHUNK, fill, 0)
    _zero_fill(zbuf, 16)
    _zero_acc_stripe(zbuf, acc_sh, sid)
    plsc.subcore_barrier()

    def chunk(j, _):
        pltpu.sync_copy(ones_v, acc_sh.at[dst_v.at[j]], add=True)
        return 0

    lax.fori_loop(0, N_CHUNKS, chunk, 0)
    plsc.subcore_barrier()
    _write_stripe(acc_sh, out_hbm, cid, sid, 16)


def _make_agg(d):
    @functools.partial(
        pl.kernel,
        mesh=plsc.VectorSubcoreMesh(**_MESH),
        out_type=jax.ShapeDtypeStruct((N_PAD, 128), jnp.float32),
        scratch_types=[
            pltpu.VMEM((N_CHUNKS, CHUNK), jnp.int32),
            pltpu.VMEM((N_CHUNKS, CHUNK), jnp.int32),
            pltpu.VMEM((CHUNK, d), jnp.float32),
            pltpu.VMEM((CHUNK, d), jnp.float32),
            pltpu.VMEM((ZROWS, d), jnp.float32),
            pltpu.VMEM_SHARED((N_ACC, d), jnp.float32),
            pltpu.SemaphoreType.DMA,
            pltpu.SemaphoreType.DMA,
        ],
        compiler_params=_SC_PARAMS,
    )
    def agg(hs_hbm, edges_hbm, out_hbm, src_v, dst_v, rows0, rows1,
            zbuf, acc_sh, sem0, sem1):
        cid = lax.axis_index("c")
        sid = lax.axis_index("s")
        wid = _wid()
        pltpu.sync_copy(edges_hbm.at[0, wid], src_v)
        pltpu.sync_copy(edges_hbm.at[1, wid], dst_v)
        _zero_fill(zbuf, d)
        _zero_acc_stripe(zbuf, acc_sh, sid)
        plsc.subcore_barrier()

        def gather(j, buf, sem):
            pltpu.async_copy(hs_hbm.at[src_v.at[j]], buf, sem)

        def wait(j, buf, sem):
            pltpu.make_async_copy(hs_hbm.at[src_v.at[j]], buf, sem).wait()

        def scat(j, buf):
            pltpu.sync_copy(buf, acc_sh.at[dst_v.at[j]], add=True)

        gather(0, rows0, sem0)
        gather(1, rows1, sem1)

        def pair(jj, _):
            j0 = 2 * jj
            j1 = j0 + 1
            wait(j0, rows0, sem0)
            scat(j0, rows0)
            gather(j0 + 2, rows0, sem0)
            wait(j1, rows1, sem1)
            scat(j1, rows1)
            gather(j1 + 2, rows1, sem1)
            return 0

        lax.fori_loop(0, N_CHUNKS // 2 - 1, pair, 0)
        t0 = N_CHUNKS - 2
        wait(t0, rows0, sem0)
        scat(t0, rows0)
        wait(t0 + 1, rows1, sem1)
        scat(t0 + 1, rows1)

        plsc.subcore_barrier()
        _write_stripe(acc_sh, out_hbm, cid, sid, d)

    return agg


_agg64 = _make_agg(64)
_agg32 = _make_agg(32)


BLK = 2000
_GRID = N_NODES // BLK


def _row_spec(cols):
    return pl.BlockSpec((BLK, cols), lambda i: (i, 0))


def _full_spec(r, c):
    return pl.BlockSpec((r, c), lambda i: (0, 0))


def _mm1_body(x_ref, w1_ref, degp_ref, hs_ref, dinv_ref):
    deg = degp_ref[:, 0:1] + degp_ref[:, 16:17] + 1.0
    dinv = lax.rsqrt(deg)
    h = jnp.dot(x_ref[...], w1_ref[...], preferred_element_type=jnp.float32)
    hs_ref[...] = h * dinv
    dinv_ref[...] = dinv


def _mm2_body(aggp_ref, hs1_ref, dinv_ref, b1_ref, w2_ref, hs2_ref):
    dinv = dinv_ref[...]
    agg = aggp_ref[:, 0:64] + aggp_ref[:, 64:128] + hs1_ref[...]
    x2 = jnp.maximum(dinv * agg + b1_ref[...], 0.0)
    hs2_ref[...] = dinv * jnp.dot(x2, w2_ref[...],
                                  preferred_element_type=jnp.float32)


def _mm3_body(aggp_ref, hs2_ref, dinv_ref, b2_ref, fcw_ref, fcb_ref,
              emb_ref, log_ref):
    dinv = dinv_ref[...]
    agg = aggp_ref[:, 0:32] + aggp_ref[:, 32:64] + hs2_ref[...]
    emb = jnp.maximum(dinv * agg + b2_ref[...], 0.0)
    emb_ref[...] = emb
    log_ref[...] = jnp.dot(emb, fcw_ref[...],
                           preferred_element_type=jnp.float32) + fcb_ref[...]


@jax.jit
def kernel(x, edge_index, W1, b1, W2, b2, fcW, fcb):
    ei = edge_index.astype(jnp.int32).reshape(2, NW, N_EDGES // NW)
    pad_src = jnp.broadcast_to(
        jnp.arange(N_PAD_W, dtype=jnp.int32) * 41 % N_NODES, (NW, N_PAD_W))
    pad_dst = (N_PAD
               + jnp.arange(NW, dtype=jnp.int32)[:, None] * N_PAD_W
               + jnp.arange(N_PAD_W, dtype=jnp.int32)[None, :])
    edges = jnp.stack([
        jnp.concatenate([ei[0], pad_src], axis=1),
        jnp.concatenate([ei[1], pad_dst], axis=1),
    ]).reshape(2, NW, N_CHUNKS, CHUNK)

    degp = _deg(edges)

    hs1, dinv = pl.pallas_call(
        _mm1_body,
        grid=(_GRID,),
        in_specs=[_row_spec(128), _full_spec(128, 64), _row_spec(128)],
        out_specs=[_row_spec(64), _row_spec(1)],
        out_shape=[
            jax.ShapeDtypeStruct((N_NODES, 64), jnp.float32),
            jax.ShapeDtypeStruct((N_NODES, 1), jnp.float32),
        ],
    )(x, W1, degp)

    aggp1 = _agg64(hs1, edges)

    hs2 = pl.pallas_call(
        _mm2_body,
        grid=(_GRID,),
        in_specs=[_row_spec(128), _row_spec(64), _row_spec(1),
                  _full_spec(1, 64), _full_spec(64, 32)],
        out_specs=_row_spec(32),
        out_shape=jax.ShapeDtypeStruct((N_NODES, 32), jnp.float32),
    )(aggp1, hs1, dinv, b1.reshape(1, -1), W2)

    aggp2 = _agg32(hs2, edges)

    emb, logits = pl.pallas_call(
        _mm3_body,
        grid=(_GRID,),
        in_specs=[_row_spec(128), _row_spec(32), _row_spec(1),
                  _full_spec(1, 32), _full_spec(32, 40), _full_spec(1, 40)],
        out_specs=[_row_spec(32), _row_spec(40)],
        out_shape=[
            jax.ShapeDtypeStruct((N_NODES, 32), jnp.float32),
            jax.ShapeDtypeStruct((N_NODES, 40), jnp.float32),
        ],
    )(aggp2, hs2, dinv, b2.reshape(1, -1), fcW, fcb.reshape(1, -1))

    return emb, logits


# speedup vs baseline: 2.1138x; 1.0954x over previous
import functools

import jax
import jax.numpy as jnp
from jax import lax
from jax.experimental import pallas as pl
from jax.experimental.pallas import tpu as pltpu
from jax.experimental.pallas import tpu_sc as plsc

N_NODES = 10000
N_PAD = 10240
N_EDGES = 320000
NC = 2
NS = 16
NW = NC * NS
E_PER_W = N_EDGES // NW
CHUNK = 128
N_FULL = E_PER_W // CHUNK
TAIL = E_PER_W - N_FULL * CHUNK
ROWS_PER_TILE = N_PAD // NS
ZROWS = 128

_MESH = dict(core_axis_name="c", subcore_axis_name="s")
_SC_PARAMS = pltpu.CompilerParams(use_tc_tiling_on_sc=False)


def _wid():
    return lax.axis_index("s") * NC + lax.axis_index("c")


def _zero_fill(zbuf, ncols):
    zero = jnp.zeros((16,), jnp.float32)

    def body(j, _):
        for c in range(ncols // 16):
            zbuf[j, pl.ds(c * 16, 16)] = zero
        return 0

    lax.fori_loop(0, ZROWS, body, 0)


def _zero_acc_stripe(zbuf, acc_sh, sid):
    base = sid * ROWS_PER_TILE
    for k in range(ROWS_PER_TILE // ZROWS):
        pltpu.sync_copy(zbuf, acc_sh.at[pl.ds(base + k * ZROWS, ZROWS)])


def _write_stripe(acc_sh, out_hbm, cid, sid, d):
    base = sid * ROWS_PER_TILE
    pltpu.sync_copy(acc_sh.at[pl.ds(base, ROWS_PER_TILE)],
                    out_hbm.at[pl.ds(base, ROWS_PER_TILE), pl.ds(d * cid, d)])


@functools.partial(
    pl.kernel,
    mesh=plsc.VectorSubcoreMesh(**_MESH),
    out_type=jax.ShapeDtypeStruct((N_PAD, 128), jnp.float32),
    scratch_types=[
        pltpu.VMEM((E_PER_W,), jnp.int32),
        pltpu.VMEM((CHUNK, 16), jnp.float32),
        pltpu.VMEM((ZROWS, 16), jnp.float32),
        pltpu.VMEM_SHARED((N_PAD, 16), jnp.float32),
    ],
    compiler_params=_SC_PARAMS,
)
def _deg(edges_hbm, out_hbm, dst_v, ones_v, zbuf, acc_sh):
    cid = lax.axis_index("c")
    sid = lax.axis_index("s")
    pltpu.sync_copy(edges_hbm.at[1, pl.ds(_wid() * E_PER_W, E_PER_W)], dst_v)

    one = jnp.full((16,), 1.0, jnp.float32)

    def fill(j, _):
        ones_v[j, :] = one
        return 0

    lax.fori_loop(0, CHUNK, fill, 0)
    _zero_fill(zbuf, 16)
    _zero_acc_stripe(zbuf, acc_sh, sid)
    plsc.subcore_barrier()

    def chunk(j, _):
        pltpu.sync_copy(ones_v, acc_sh.at[dst_v.at[pl.ds(j * CHUNK, CHUNK)]],
                        add=True)
        return 0

    lax.fori_loop(0, N_FULL, chunk, 0)
    pltpu.sync_copy(ones_v.at[pl.ds(0, TAIL)],
                    acc_sh.at[dst_v.at[pl.ds(N_FULL * CHUNK, TAIL)]],
                    add=True)
    plsc.subcore_barrier()
    _write_stripe(acc_sh, out_hbm, cid, sid, 16)


def _make_agg(d):
    @functools.partial(
        pl.kernel,
        mesh=plsc.VectorSubcoreMesh(**_MESH),
        out_type=jax.ShapeDtypeStruct((N_PAD, 128), jnp.float32),
        scratch_types=[
            pltpu.VMEM((E_PER_W,), jnp.int32),
            pltpu.VMEM((E_PER_W,), jnp.int32),
            pltpu.VMEM((CHUNK, d), jnp.float32),
            pltpu.VMEM((CHUNK, d), jnp.float32),
            pltpu.VMEM((ZROWS, d), jnp.float32),
            pltpu.VMEM_SHARED((N_PAD, d), jnp.float32),
            pltpu.SemaphoreType.DMA,
            pltpu.SemaphoreType.DMA,
        ],
        compiler_params=_SC_PARAMS,
    )
    def agg(hs_hbm, edges_hbm, out_hbm, src_v, dst_v, rows0, rows1,
            zbuf, acc_sh, sem0, sem1):
        cid = lax.axis_index("c")
        sid = lax.axis_index("s")
        base_e = _wid() * E_PER_W
        pltpu.sync_copy(edges_hbm.at[0, pl.ds(base_e, E_PER_W)], src_v)
        pltpu.sync_copy(edges_hbm.at[1, pl.ds(base_e, E_PER_W)], dst_v)
        _zero_fill(zbuf, d)
        _zero_acc_stripe(zbuf, acc_sh, sid)
        plsc.subcore_barrier()

        def gather(j, n, buf, sem):
            pltpu.async_copy(hs_hbm.at[src_v.at[pl.ds(j * CHUNK, n)]],
                             buf, sem)

        def wait(j, n, buf, sem):
            pltpu.make_async_copy(hs_hbm.at[src_v.at[pl.ds(j * CHUNK, n)]],
                                  buf, sem).wait()

        def scat(j, n, buf):
            pltpu.sync_copy(buf, acc_sh.at[dst_v.at[pl.ds(j * CHUNK, n)]],
                            add=True)

        gather(0, CHUNK, rows0, sem0)
        gather(1, CHUNK, rows1, sem1)

        def pair(jj, _):
            j0 = 2 * jj
            j1 = j0 + 1
            wait(j0, CHUNK, rows0, sem0)
            scat(j0, CHUNK, rows0)
            gather(j0 + 2, CHUNK, rows0, sem0)
            wait(j1, CHUNK, rows1, sem1)
            scat(j1, CHUNK, rows1)
            gather(j1 + 2, CHUNK, rows1, sem1)
            return 0

        lax.fori_loop(0, N_FULL // 2 - 1, pair, 0)
        t0 = N_FULL - 2
        wait(t0, CHUNK, rows0, sem0)
        scat(t0, CHUNK, rows0)
        tail_buf = rows1
        wait(t0 + 1, CHUNK, rows1, sem1)
        scat(t0 + 1, CHUNK, rows1)
        pltpu.async_copy(hs_hbm.at[src_v.at[pl.ds(N_FULL * CHUNK, TAIL)]],
                         tail_buf.at[pl.ds(0, TAIL)], sem0)
        pltpu.make_async_copy(hs_hbm.at[src_v.at[pl.ds(N_FULL * CHUNK, TAIL)]],
                              tail_buf.at[pl.ds(0, TAIL)], sem0).wait()
        pltpu.sync_copy(tail_buf.at[pl.ds(0, TAIL)],
                        acc_sh.at[dst_v.at[pl.ds(N_FULL * CHUNK, TAIL)]],
                        add=True)

        plsc.subcore_barrier()
        _write_stripe(acc_sh, out_hbm, cid, sid, d)

    return agg


_agg64 = _make_agg(64)
_agg32 = _make_agg(32)


BLK = 2000
_GRID = N_NODES // BLK


def _row_spec(cols):
    return pl.BlockSpec((BLK, cols), lambda i: (i, 0))


def _full_spec(r, c):
    return pl.BlockSpec((r, c), lambda i: (0, 0))


def _mm1_body(x_ref, w1_ref, degp_ref, hs_ref, dinv_ref):
    deg = degp_ref[:, 0:1] + degp_ref[:, 16:17] + 1.0
    dinv = lax.rsqrt(deg)
    h = jnp.dot(x_ref[...], w1_ref[...], preferred_element_type=jnp.float32)
    hs_ref[...] = h * dinv
    dinv_ref[...] = dinv


def _mm2_body(aggp_ref, hs1_ref, dinv_ref, b1_ref, w2_ref, hs2_ref):
    dinv = dinv_ref[...]
    agg = aggp_ref[:, 0:64] + aggp_ref[:, 64:128] + hs1_ref[...]
    x2 = jnp.maximum(dinv * agg + b1_ref[...], 0.0)
    hs2_ref[...] = dinv * jnp.dot(x2, w2_ref[...],
                                  preferred_element_type=jnp.float32)


def _mm3_body(aggp_ref, hs2_ref, dinv_ref, b2_ref, fcw_ref, fcb_ref,
              emb_ref, log_ref):
    dinv = dinv_ref[...]
    agg = aggp_ref[:, 0:32] + aggp_ref[:, 32:64] + hs2_ref[...]
    emb = jnp.maximum(dinv * agg + b2_ref[...], 0.0)
    emb_ref[...] = emb
    log_ref[...] = jnp.dot(emb, fcw_ref[...],
                           preferred_element_type=jnp.float32) + fcb_ref[...]


@jax.jit
def kernel(x, edge_index, W1, b1, W2, b2, fcW, fcb):
    edges = edge_index.astype(jnp.int32)

    degp = _deg(edges)

    hs1, dinv = pl.pallas_call(
        _mm1_body,
        grid=(_GRID,),
        in_specs=[_row_spec(128), _full_spec(128, 64), _row_spec(128)],
        out_specs=[_row_spec(64), _row_spec(1)],
        out_shape=[
            jax.ShapeDtypeStruct((N_NODES, 64), jnp.float32),
            jax.ShapeDtypeStruct((N_NODES, 1), jnp.float32),
        ],
    )(x, W1, degp)

    aggp1 = _agg64(hs1, edges)

    hs2 = pl.pallas_call(
        _mm2_body,
        grid=(_GRID,),
        in_specs=[_row_spec(128), _row_spec(64), _row_spec(1),
                  _full_spec(1, 64), _full_spec(64, 32)],
        out_specs=_row_spec(32),
        out_shape=jax.ShapeDtypeStruct((N_NODES, 32), jnp.float32),
    )(aggp1, hs1, dinv, b1.reshape(1, -1), W2)

    aggp2 = _agg32(hs2, edges)

    emb, logits = pl.pallas_call(
        _mm3_body,
        grid=(_GRID,),
        in_specs=[_row_spec(128), _row_spec(32), _row_spec(1),
                  _full_spec(1, 32), _full_spec(32, 40), _full_spec(1, 40)],
        out_specs=[_row_spec(32), _row_spec(40)],
        out_shape=[
            jax.ShapeDtypeStruct((N_NODES, 32), jnp.float32),
            jax.ShapeDtypeStruct((N_NODES, 40), jnp.float32),
        ],
    )(aggp2, hs2, dinv, b2.reshape(1, -1), fcW, fcb.reshape(1, -1))

    return emb, logits
